# baseline (speedup 1.0000x reference)
import jax
import jax.numpy as jnp
from jax import lax
from jax.experimental import pallas as pl
from jax.experimental.pallas import tpu as pltpu
from jax.experimental.pallas import tpu_sc as plsc

N = 100000
EMB_DIM = 64
D_FEAT = 128
OUT_DIM = EMB_DIM + D_FEAT
BM = 2048
HB = BM // 2
NG = -(-N // BM)
NP = NG * BM
PP = NP // 2

RP = 256
NW = 32
DEPTH = 3
CHUNKS = [6, 9, 11, 11, 12]


def _sc_gather_body(b0, nbp, maxj, ids_hbm, table_hbm, emb2_hbm,
                    idx_v, lo_v, hi_v, *sems):
    NBP, MAXJ = nbp, maxj
    SB = BM // RP
    wid = lax.axis_index("s") * 2 + lax.axis_index("c")

    def sem(p, k):
        return sems[p * 5 + k]

    def blk(j):
        return wid + j * NW

    def lo_base(j):
        g = b0 * (HB // RP) + blk(j)
        return (g // (HB // RP)) * BM + (g % (HB // RP)) * RP

    def front(j):
        p = j % DEPTH

        @pl.when(blk(j) < NBP)
        def _():
            lob = lo_base(j)
            pltpu.make_async_copy(
                ids_hbm.at[pl.ds(lob, RP)], idx_v.at[p, pl.ds(0, RP)],
                sem(p, 0)).start()
            pltpu.make_async_copy(
                ids_hbm.at[pl.ds(lob + HB, RP)], idx_v.at[p, pl.ds(RP, RP)],
                sem(p, 0)).start()

    def mid(j):
        p = j % DEPTH

        @pl.when(blk(j) < NBP)
        def _():
            lob = lo_base(j)
            pltpu.make_async_copy(
                ids_hbm.at[pl.ds(lob, RP)], idx_v.at[p, pl.ds(0, RP)],
                sem(p, 0)).wait()
            pltpu.make_async_copy(
                ids_hbm.at[pl.ds(lob + HB, RP)], idx_v.at[p, pl.ds(RP, RP)],
                sem(p, 0)).wait()
            pltpu.make_async_copy(
                table_hbm.at[idx_v.at[p, pl.ds(0, RP)]],
                lo_v.at[p], sem(p, 1)).start()
            pltpu.make_async_copy(
                table_hbm.at[idx_v.at[p, pl.ds(RP, RP)]],
                hi_v.at[p], sem(p, 2)).start()

    def back(j):
        p = j % DEPTH

        @pl.when(blk(j) < NBP)
        def _():
            base = blk(j) * RP
            pltpu.make_async_copy(
                table_hbm.at[idx_v.at[p, pl.ds(0, RP)]],
                lo_v.at[p], sem(p, 1)).wait()
            pltpu.make_async_copy(
                table_hbm.at[idx_v.at[p, pl.ds(RP, RP)]],
                hi_v.at[p], sem(p, 2)).wait()
            pltpu.make_async_copy(
                lo_v.at[p],
                emb2_hbm.at[pl.ds(base, RP), pl.ds(0, EMB_DIM)],
                sem(p, 3)).start()
            pltpu.make_async_copy(
                hi_v.at[p],
                emb2_hbm.at[pl.ds(base, RP), pl.ds(EMB_DIM, EMB_DIM)],
                sem(p, 4)).start()

    def drain(j):
        p = j % DEPTH

        @pl.when(blk(j) < NBP)
        def _():
            base = blk(j) * RP
            pltpu.make_async_copy(
                lo_v.at[p],
                emb2_hbm.at[pl.ds(base, RP), pl.ds(0, EMB_DIM)],
                sem(p, 3)).wait()
            pltpu.make_async_copy(
                hi_v.at[p],
                emb2_hbm.at[pl.ds(base, RP), pl.ds(EMB_DIM, EMB_DIM)],
                sem(p, 4)).wait()

    for step in range(MAXJ + DEPTH):
        jd = step - DEPTH
        if 0 <= jd < MAXJ:
            drain(jd)
        if step < MAXJ:
            front(step)
        jm = step - 1
        if 0 <= jm < MAXJ:
            mid(jm)
        jb = step - 2
        if 0 <= jb < MAXJ:
            back(jb)


def _sc_gather(b0, nb, ids_p, embed_weight):
    import functools
    pp = nb * HB
    nbp = pp // RP
    maxj = -(-nbp // NW)
    mesh = plsc.VectorSubcoreMesh(core_axis_name="c", subcore_axis_name="s")
    return pl.kernel(
        functools.partial(_sc_gather_body, b0, nbp, maxj),
        mesh=mesh,
        out_type=jax.ShapeDtypeStruct((pp, 2 * EMB_DIM), jnp.float32),
        scratch_types=[
            pltpu.VMEM((DEPTH, 2 * RP), jnp.int32),
            pltpu.VMEM((DEPTH, RP, EMB_DIM), jnp.float32),
            pltpu.VMEM((DEPTH, RP, EMB_DIM), jnp.float32),
        ] + [pltpu.SemaphoreType.DMA] * (DEPTH * 5),
        compiler_params=pltpu.CompilerParams(use_tc_tiling_on_sc=False),
    )(ids_p, embed_weight)


def _tc_concat_body(emb2_ref, feats_ref, *rest):
    out_ref = rest[-1]
    e = emb2_ref[...]
    f = feats_ref[...]
    et = e.T
    emb_part = jnp.concatenate(
        [et[:EMB_DIM, :], et[EMB_DIM:, :]], axis=1)
    out_ref[...] = jnp.concatenate(
        [emb_part, f.T], axis=0)


def _tc_concat(b0, nb, emb2_c, feats, out_prev):
    in_specs = [
        pl.BlockSpec((HB, 2 * EMB_DIM), lambda i: (i, 0)),
        pl.BlockSpec((BM, D_FEAT), lambda i: (b0 + i, 0)),
    ]
    args = [emb2_c, feats]
    alias = {}
    if out_prev is not None:
        in_specs.append(pl.BlockSpec(memory_space=pl.ANY))
        args.append(out_prev)
        alias = {2: 0}
    return pl.pallas_call(
        _tc_concat_body,
        grid=(nb,),
        in_specs=in_specs,
        out_specs=pl.BlockSpec((OUT_DIM, BM), lambda i: (0, b0 + i)),
        out_shape=jax.ShapeDtypeStruct((OUT_DIM, N), jnp.float32),
        input_output_aliases=alias,
    )(*args)


@jax.jit
def _feature_prep(ids, feats, embed_weight):
    ids_p = jnp.pad(ids, (0, NP - N))
    emb2 = []
    b0 = 0
    for nb in CHUNKS:
        emb2.append(_sc_gather(b0, nb, ids_p, embed_weight))
        b0 += nb
    out_t = None
    b0 = 0
    for c, nb in enumerate(CHUNKS):
        out_t = _tc_concat(b0, nb, emb2[c], feats, out_t)
        b0 += nb
    return out_t.T


def kernel(ids, feats, embed_weight):
    return _feature_prep(ids.astype(jnp.int32), feats, embed_weight)


# speedup vs baseline: 1.0118x; 1.0118x over previous
import jax
import jax.numpy as jnp
from jax import lax
from jax.experimental import pallas as pl
from jax.experimental.pallas import tpu as pltpu
from jax.experimental.pallas import tpu_sc as plsc

N = 100000
EMB_DIM = 64
D_FEAT = 128
OUT_DIM = EMB_DIM + D_FEAT
BM = 2048
HB = BM // 2
NG = -(-N // BM)
NP = NG * BM
PP = NP // 2

RP = 256
NW = 32
DEPTH = 3
CHUNKS = [5, 8, 12, 12, 12]


def _sc_gather_body(b0, nbp, maxj, ids_hbm, table_hbm, emb2_hbm,
                    idx_v, lo_v, hi_v, *sems):
    NBP, MAXJ = nbp, maxj
    SB = BM // RP
    wid = lax.axis_index("s") * 2 + lax.axis_index("c")

    def sem(p, k):
        return sems[p * 5 + k]

    def blk(j):
        return wid + j * NW

    def lo_base(j):
        g = b0 * (HB // RP) + blk(j)
        return (g // (HB // RP)) * BM + (g % (HB // RP)) * RP

    def front(j):
        p = j % DEPTH

        @pl.when(blk(j) < NBP)
        def _():
            lob = lo_base(j)
            pltpu.make_async_copy(
                ids_hbm.at[pl.ds(lob, RP)], idx_v.at[p, pl.ds(0, RP)],
                sem(p, 0)).start()
            pltpu.make_async_copy(
                ids_hbm.at[pl.ds(lob + HB, RP)], idx_v.at[p, pl.ds(RP, RP)],
                sem(p, 0)).start()

    def mid(j):
        p = j % DEPTH

        @pl.when(blk(j) < NBP)
        def _():
            lob = lo_base(j)
            pltpu.make_async_copy(
                ids_hbm.at[pl.ds(lob, RP)], idx_v.at[p, pl.ds(0, RP)],
                sem(p, 0)).wait()
            pltpu.make_async_copy(
                ids_hbm.at[pl.ds(lob + HB, RP)], idx_v.at[p, pl.ds(RP, RP)],
                sem(p, 0)).wait()
            pltpu.make_async_copy(
                table_hbm.at[idx_v.at[p, pl.ds(0, RP)]],
                lo_v.at[p], sem(p, 1)).start()
            pltpu.make_async_copy(
                table_hbm.at[idx_v.at[p, pl.ds(RP, RP)]],
                hi_v.at[p], sem(p, 2)).start()

    def back(j):
        p = j % DEPTH

        @pl.when(blk(j) < NBP)
        def _():
            base = blk(j) * RP
            pltpu.make_async_copy(
                table_hbm.at[idx_v.at[p, pl.ds(0, RP)]],
                lo_v.at[p], sem(p, 1)).wait()
            pltpu.make_async_copy(
                table_hbm.at[idx_v.at[p, pl.ds(RP, RP)]],
                hi_v.at[p], sem(p, 2)).wait()
            pltpu.make_async_copy(
                lo_v.at[p],
                emb2_hbm.at[pl.ds(base, RP), pl.ds(0, EMB_DIM)],
                sem(p, 3)).start()
            pltpu.make_async_copy(
                hi_v.at[p],
                emb2_hbm.at[pl.ds(base, RP), pl.ds(EMB_DIM, EMB_DIM)],
                sem(p, 4)).start()

    def drain(j):
        p = j % DEPTH

        @pl.when(blk(j) < NBP)
        def _():
            base = blk(j) * RP
            pltpu.make_async_copy(
                lo_v.at[p],
                emb2_hbm.at[pl.ds(base, RP), pl.ds(0, EMB_DIM)],
                sem(p, 3)).wait()
            pltpu.make_async_copy(
                hi_v.at[p],
                emb2_hbm.at[pl.ds(base, RP), pl.ds(EMB_DIM, EMB_DIM)],
                sem(p, 4)).wait()

    for step in range(MAXJ + DEPTH):
        jd = step - DEPTH
        if 0 <= jd < MAXJ:
            drain(jd)
        if step < MAXJ:
            front(step)
        jm = step - 1
        if 0 <= jm < MAXJ:
            mid(jm)
        jb = step - 2
        if 0 <= jb < MAXJ:
            back(jb)


def _sc_gather(b0, nb, ids_p, embed_weight):
    import functools
    pp = nb * HB
    nbp = pp // RP
    maxj = -(-nbp // NW)
    mesh = plsc.VectorSubcoreMesh(core_axis_name="c", subcore_axis_name="s")
    return pl.kernel(
        functools.partial(_sc_gather_body, b0, nbp, maxj),
        mesh=mesh,
        out_type=jax.ShapeDtypeStruct((pp, 2 * EMB_DIM), jnp.float32),
        scratch_types=[
            pltpu.VMEM((DEPTH, 2 * RP), jnp.int32),
            pltpu.VMEM((DEPTH, RP, EMB_DIM), jnp.float32),
            pltpu.VMEM((DEPTH, RP, EMB_DIM), jnp.float32),
        ] + [pltpu.SemaphoreType.DMA] * (DEPTH * 5),
        compiler_params=pltpu.CompilerParams(use_tc_tiling_on_sc=False),
    )(ids_p, embed_weight)


def _tc_concat_body(emb2_ref, feats_ref, *rest):
    out_ref = rest[-1]
    e = emb2_ref[...]
    f = feats_ref[...]
    et = e.T
    emb_part = jnp.concatenate(
        [et[:EMB_DIM, :], et[EMB_DIM:, :]], axis=1)
    out_ref[...] = jnp.concatenate(
        [emb_part, f.T], axis=0)


def _tc_concat(b0, nb, emb2_c, feats, out_prev):
    in_specs = [
        pl.BlockSpec((HB, 2 * EMB_DIM), lambda i: (i, 0)),
        pl.BlockSpec((BM, D_FEAT), lambda i: (b0 + i, 0)),
    ]
    args = [emb2_c, feats]
    alias = {}
    if out_prev is not None:
        in_specs.append(pl.BlockSpec(memory_space=pl.ANY))
        args.append(out_prev)
        alias = {2: 0}
    return pl.pallas_call(
        _tc_concat_body,
        grid=(nb,),
        in_specs=in_specs,
        out_specs=pl.BlockSpec((OUT_DIM, BM), lambda i: (0, b0 + i)),
        out_shape=jax.ShapeDtypeStruct((OUT_DIM, N), jnp.float32),
        input_output_aliases=alias,
    )(*args)


@jax.jit
def _feature_prep(ids, feats, embed_weight):
    ids_p = jnp.pad(ids, (0, NP - N))
    emb2 = []
    b0 = 0
    for nb in CHUNKS:
        emb2.append(_sc_gather(b0, nb, ids_p, embed_weight))
        b0 += nb
    out_t = None
    b0 = 0
    for c, nb in enumerate(CHUNKS):
        out_t = _tc_concat(b0, nb, emb2[c], feats, out_t)
        b0 += nb
    return out_t.T


def kernel(ids, feats, embed_weight):
    return _feature_prep(ids.astype(jnp.int32), feats, embed_weight)
